# baseline (speedup 1.0000x reference)
import dataclasses
import functools

import jax
import jax.numpy as jnp
import numpy as np
from jax import lax
from jax.experimental import pallas as pl
from jax.experimental.pallas import tpu as pltpu
from jax.experimental.pallas import tpu_sc as plsc

MAX_POS = 200
EMBED_DIM = 64
LANES = 16

NUM_CORES = 2
NUM_SUBCORES = 16
NUM_WORKERS = NUM_CORES * NUM_SUBCORES

CHUNK = 200
GATHER_W = 100
GATHERS = CHUNK // GATHER_W
NBUF = 4


def _pos_enc() -> np.ndarray:
    pos = np.arange(MAX_POS)[:, np.newaxis]
    i = np.arange(EMBED_DIM)[np.newaxis, :]
    angle_rates = 1 / np.power(10000, 2 * (i // 2) / np.float32(EMBED_DIM))
    angle_rads = pos * angle_rates
    angle_rads[:, 0::2] = np.sin(angle_rads[:, 0::2])
    angle_rads[:, 1::2] = np.cos(angle_rads[:, 1::2])
    return angle_rads.astype(np.float32)


def _make_sc_embed(batch: int, seq: int):
    seq_per_w = batch // NUM_WORKERS
    assert seq == CHUNK and seq_per_w % NBUF == 0
    mesh = plsc.VectorSubcoreMesh(core_axis_name="c", subcore_axis_name="s")
    cp = pltpu.CompilerParams()
    if "needs_layout_passes" in pltpu.CompilerParams.__dataclass_fields__:
        cp = dataclasses.replace(cp, needs_layout_passes=False)
    if "use_tc_tiling_on_sc" in pltpu.CompilerParams.__dataclass_fields__:
        cp = dataclasses.replace(cp, use_tc_tiling_on_sc=False)

    @functools.partial(
        pl.kernel,
        out_type=jax.ShapeDtypeStruct((batch, seq, EMBED_DIM), jnp.float32),
        mesh=mesh,
        compiler_params=cp,
        scratch_types=[
            pltpu.VMEM((NBUF, GATHERS, GATHER_W), jnp.int32),
            pltpu.VMEM((NBUF, CHUNK), jnp.float32),
            pltpu.VMEM((NBUF, CHUNK, EMBED_DIM), jnp.float32),
            pltpu.VMEM((MAX_POS, EMBED_DIM), jnp.float32),
            pltpu.SemaphoreType.DMA((NBUF,)),
            pltpu.SemaphoreType.DMA((NBUF,)),
            pltpu.SemaphoreType.DMA((NBUF,)),
        ],
    )
    def sc_embed(table_hbm, idx_hbm, dur_hbm, pos_hbm, out_hbm,
                 idx_v, dur_v, rows_v, pos_v, sem_in, sem_g, sem_out):
        wid = lax.axis_index("s") * NUM_CORES + lax.axis_index("c")
        base = wid * seq_per_w
        pltpu.sync_copy(pos_hbm, pos_v)

        def issue_in(c, b):
            pltpu.async_copy(idx_hbm.at[base + c], idx_v.at[b], sem_in.at[b])
            pltpu.async_copy(dur_hbm.at[base + c], dur_v.at[b], sem_in.at[b])

        def wait_in(b):
            pltpu.make_async_copy(
                idx_hbm.at[0], idx_v.at[b], sem_in.at[b]).wait()
            pltpu.make_async_copy(
                dur_hbm.at[0], dur_v.at[b], sem_in.at[b]).wait()

        def issue_gather(b):
            for j in range(GATHERS):
                pltpu.async_copy(
                    table_hbm.at[idx_v.at[b, j]],
                    rows_v.at[b].at[pl.ds(j * GATHER_W, GATHER_W)],
                    sem_g.at[b])

        def wait_gather(b):
            for j in range(GATHERS):
                pltpu.make_async_copy(
                    table_hbm.at[idx_v.at[b, j]],
                    rows_v.at[b].at[pl.ds(j * GATHER_W, GATHER_W)],
                    sem_g.at[b]).wait()

        def issue_out(c, b):
            pltpu.async_copy(rows_v.at[b], out_hbm.at[base + c], sem_out.at[b])

        def wait_out(b):
            pltpu.make_async_copy(
                rows_v.at[b], out_hbm.at[0], sem_out.at[b]).wait()

        issue_in(0, 0)
        issue_in(1, 1)
        wait_in(0)
        issue_gather(0)

        @pl.loop(0, seq_per_w, step=NBUF)
        def _ring(cc):
            for b in range(NBUF):
                c = cc + b
                b1, b2 = (b + 1) % NBUF, (b + 2) % NBUF

                @pl.when(c + 1 < seq_per_w)
                def _():
                    wait_in(b1)

                @pl.when(c >= NBUF - 1)
                def _():
                    wait_out(b1)

                @pl.when(c + 1 < seq_per_w)
                def _():
                    issue_gather(b1)

                @pl.when(c + 2 < seq_per_w)
                def _():
                    issue_in(c + 2, b2)

                wait_gather(b)

                @plsc.parallel_loop(0, CHUNK, 1, unroll=4)
                def _tok(t):
                    durb = plsc.load_gather(
                        dur_v.at[b], [lax.broadcast(t, (LANES,))])
                    for d in range(EMBED_DIM // LANES):
                        sl = pl.ds(d * LANES, LANES)
                        rows_v[b, t, sl] = (
                            rows_v[b, t, sl] + pos_v[t, sl] + durb)

                issue_out(c, b)

        for k in range(seq_per_w - NBUF + 1, seq_per_w):
            wait_out(k % NBUF)

    return sc_embed


def kernel(x_in, table):
    batch, seq, _ = x_in.shape
    notes = x_in[:, :, 0].reshape(batch, GATHERS, GATHER_W)
    dur = x_in[:, :, 1].astype(jnp.float32)
    pos = jnp.asarray(_pos_enc())
    return _make_sc_embed(batch, seq)(table, notes, dur, pos)
